# Initial kernel scaffold; baseline (speedup 1.0000x reference)
import jax
import jax.numpy as jnp
from jax.experimental import pallas as pl


def kernel(x, edge_index, w, W1, Wfc, ln_g, ln_b, W2, W3):
    raise NotImplementedError("write your pallas kernel here")


# speedup vs baseline: 1.0295x; 1.0295x over previous
import functools

import jax
import jax.numpy as jnp
from jax.experimental import pallas as pl
from jax.experimental.pallas import tpu as pltpu

N = 10000
D = 128
BM = 1000
GRID = N // BM


def _tc_pre_body(x_ref, w1_ref, wfc_ref, g_ref, b_ref, do_ref, di_ref,
                 h1_ref, f1s_ref, rio_ref, rsin_ref):
    x = x_ref[...]
    rs_o = jax.lax.rsqrt(jnp.maximum(do_ref[...], 1.0))
    rs_i = jax.lax.rsqrt(jnp.maximum(di_ref[...], 1.0))
    xo = x * rs_o
    h1_ref[...] = jnp.dot(xo, w1_ref[...], preferred_element_type=jnp.float32)
    f = jnp.dot(x, wfc_ref[...], preferred_element_type=jnp.float32)
    mu = jnp.mean(f, axis=-1, keepdims=True)
    var = jnp.mean((f - mu) * (f - mu), axis=-1, keepdims=True)
    f1 = jnp.maximum((f - mu) * jax.lax.rsqrt(var + 1e-5) * g_ref[...] + b_ref[...], 0.0)
    f1s_ref[...] = f1 * rs_o
    rio_ref[...] = rs_i * rs_o
    rsin_ref[...] = rs_i


def _tc_pre(x, W1, Wfc, ln_g, ln_b, deg_out, deg_in):
    return pl.pallas_call(
        _tc_pre_body,
        grid=(GRID,),
        in_specs=[
            pl.BlockSpec((BM, D), lambda i: (i, 0)),
            pl.BlockSpec((D, D), lambda i: (0, 0)),
            pl.BlockSpec((D, D), lambda i: (0, 0)),
            pl.BlockSpec((1, D), lambda i: (0, 0)),
            pl.BlockSpec((1, D), lambda i: (0, 0)),
            pl.BlockSpec((BM, 1), lambda i: (i, 0)),
            pl.BlockSpec((BM, 1), lambda i: (i, 0)),
        ],
        out_specs=[
            pl.BlockSpec((BM, D), lambda i: (i, 0)),
            pl.BlockSpec((BM, D), lambda i: (i, 0)),
            pl.BlockSpec((BM, 1), lambda i: (i, 0)),
            pl.BlockSpec((BM, 1), lambda i: (i, 0)),
        ],
        out_shape=[
            jax.ShapeDtypeStruct((N, D), jnp.float32),
            jax.ShapeDtypeStruct((N, D), jnp.float32),
            jax.ShapeDtypeStruct((N, 1), jnp.float32),
            jax.ShapeDtypeStruct((N, 1), jnp.float32),
        ],
    )(x, W1, Wfc, ln_g.reshape(1, D), ln_b.reshape(1, D), deg_out, deg_in)


def _tc_mid2_body(agga_ref, aggb_ref, f1s_ref, w2a_ref, w2b_ref, rio_ref, out_ref):
    x1s = jnp.maximum(agga_ref[...] + aggb_ref[...], 0.0) * rio_ref[...]
    out_ref[...] = (
        jnp.dot(x1s, w2a_ref[...], preferred_element_type=jnp.float32)
        + jnp.dot(f1s_ref[...], w2b_ref[...], preferred_element_type=jnp.float32))


def _tc_mid2(agga, aggb, f1s, W2a, W2b, rio):
    return pl.pallas_call(
        _tc_mid2_body,
        grid=(GRID,),
        in_specs=[
            pl.BlockSpec((BM, D), lambda i: (i, 0)),
            pl.BlockSpec((BM, D), lambda i: (i, 0)),
            pl.BlockSpec((BM, D), lambda i: (i, 0)),
            pl.BlockSpec((D, D), lambda i: (0, 0)),
            pl.BlockSpec((D, D), lambda i: (0, 0)),
            pl.BlockSpec((BM, 1), lambda i: (i, 0)),
        ],
        out_specs=pl.BlockSpec((BM, D), lambda i: (i, 0)),
        out_shape=jax.ShapeDtypeStruct((N, D), jnp.float32),
    )(agga, aggb, f1s, W2a, W2b, rio)


def _tc_mid3_body(agga_ref, aggb_ref, w3_ref, rio_ref, out_ref):
    x2s = jnp.maximum(agga_ref[...] + aggb_ref[...], 0.0) * rio_ref[...]
    out_ref[...] = jnp.dot(x2s, w3_ref[...], preferred_element_type=jnp.float32)


def _tc_mid3(agga, aggb, W3, rio):
    return pl.pallas_call(
        _tc_mid3_body,
        grid=(GRID,),
        in_specs=[
            pl.BlockSpec((BM, D), lambda i: (i, 0)),
            pl.BlockSpec((BM, D), lambda i: (i, 0)),
            pl.BlockSpec((D, D), lambda i: (0, 0)),
            pl.BlockSpec((BM, 1), lambda i: (i, 0)),
        ],
        out_specs=pl.BlockSpec((BM, D), lambda i: (i, 0)),
        out_shape=jax.ShapeDtypeStruct((N, D), jnp.float32),
    )(agga, aggb, W3, rio)


def _tc_final_body(agga_ref, aggb_ref, rsin_ref, out_ref):
    i = pl.program_id(0)
    x3 = jnp.maximum(agga_ref[...] + aggb_ref[...], 0.0) * rsin_ref[...]
    part = jnp.sum(x3, axis=0, keepdims=True)

    @pl.when(i == 0)
    def _():
        out_ref[...] = part

    @pl.when(i != 0)
    def _():
        out_ref[...] += part


def _tc_final(agga, aggb, rs_in):
    return pl.pallas_call(
        _tc_final_body,
        grid=(GRID,),
        in_specs=[
            pl.BlockSpec((BM, D), lambda i: (i, 0)),
            pl.BlockSpec((BM, D), lambda i: (i, 0)),
            pl.BlockSpec((BM, 1), lambda i: (i, 0)),
        ],
        out_specs=pl.BlockSpec((1, D), lambda i: (0, 0)),
        out_shape=jax.ShapeDtypeStruct((1, D), jnp.float32),
    )(agga, aggb, rs_in)


def _edge_aggregate(h, src, dst, w):
    m = h[src] * w[:, None]
    agg = jnp.zeros((N, D), jnp.float32).at[dst].add(m)
    return agg, jnp.zeros((N, D), jnp.float32)


def _degrees(src, dst):
    deg_out = jnp.zeros((N, 1), jnp.float32).at[src].add(1.0)
    deg_in = jnp.zeros((N, 1), jnp.float32).at[dst].add(1.0)
    return deg_out, deg_in


def kernel(x, edge_index, w, W1, Wfc, ln_g, ln_b, W2, W3):
    src, dst = edge_index[0], edge_index[1]
    deg_out, deg_in = _degrees(src, dst)
    h1, f1s, rio, rs_in = _tc_pre(x, W1, Wfc, ln_g, ln_b, deg_out, deg_in)
    agg1a, agg1b = _edge_aggregate(h1, src, dst, w)
    h2 = _tc_mid2(agg1a, agg1b, f1s, W2[:D], W2[D:], rio)
    agg2a, agg2b = _edge_aggregate(h2, src, dst, w)
    h3 = _tc_mid3(agg2a, agg2b, W3, rio)
    agg3a, agg3b = _edge_aggregate(h3, src, dst, w)
    return _tc_final(agg3a, agg3b, rs_in)


# speedup vs baseline: 2.9678x; 2.8826x over previous
import functools

import jax
import jax.numpy as jnp
from jax import lax
from jax.experimental import pallas as pl
from jax.experimental.pallas import tpu as pltpu
from jax.experimental.pallas import tpu_sc as plsc

N = 10000
D = 128
E = 320000
BM = 1000
GRID = N // BM

NC, NS, L = 2, 16, 16
NW = NC * NS
CHUNK = 128
CPW = 79
EPW = CPW * CHUNK
E_PAD = NW * EPW
N_PAD = 10240
RPW = N_PAD // NS
SINK = N

_sc_mesh = plsc.VectorSubcoreMesh(core_axis_name="c", subcore_axis_name="s")


def _sc_aggregate_body(h_hbm, src_hbm, dst_hbm, w_hbm, out_hbm,
                       acc, src_v, dst_v, w_v, rows, sem):
    cid = lax.axis_index("c")
    sid = lax.axis_index("s")
    wid = cid * NS + sid

    def zrow(i, carry):
        z = jnp.zeros((L,), jnp.float32)
        for g in range(D // L):
            rows[i, pl.ds(g * L, L)] = z
        return carry

    lax.fori_loop(0, CHUNK, zrow, 0)
    rbase = sid * RPW
    for k in range(RPW // CHUNK):
        pltpu.sync_copy(rows, acc.at[pl.ds(rbase + k * CHUNK, CHUNK)])
    plsc.subcore_barrier()

    ebase = wid * EPW

    def chunk_body(c, carry):
        off = ebase + c * CHUNK
        pltpu.sync_copy(src_hbm.at[pl.ds(off, CHUNK)], src_v)
        pltpu.sync_copy(dst_hbm.at[pl.ds(off, CHUNK)], dst_v)
        pltpu.sync_copy(w_hbm.at[pl.ds(off, CHUNK)], w_v.at[pl.ds(0, CHUNK)])
        pltpu.async_copy(h_hbm.at[src_v], rows, sem).wait()

        def mul_body(e, c2):
            ws = w_v[pl.ds(e, L)][0]
            for g in range(D // L):
                sl = pl.ds(g * L, L)
                rows[e, sl] = rows[e, sl] * ws
            return c2

        lax.fori_loop(0, CHUNK, mul_body, 0)
        pltpu.sync_copy(rows, acc.at[dst_v], add=True)
        return carry

    lax.fori_loop(0, CPW, chunk_body, 0)
    plsc.subcore_barrier()

    pltpu.sync_copy(acc.at[pl.ds(rbase, RPW)],
                    out_hbm.at[cid, pl.ds(rbase, RPW)])


_sc_aggregate = pl.kernel(
    _sc_aggregate_body,
    out_type=jax.ShapeDtypeStruct((NC, N_PAD, D), jnp.float32),
    mesh=_sc_mesh,
    scratch_types=[
        pltpu.VMEM_SHARED((N_PAD, D), jnp.float32),
        pltpu.VMEM((CHUNK,), jnp.int32),
        pltpu.VMEM((CHUNK,), jnp.int32),
        pltpu.VMEM((CHUNK + L,), jnp.float32),
        pltpu.VMEM((CHUNK, D), jnp.float32),
        pltpu.SemaphoreType.DMA,
    ],
)


OUT_LANE = 0
IN_LANE = 64


def _sc_degrees_body(src_hbm, dst_hbm, deg_hbm,
                     acc, src_v, dst_v, vsrc, vdst):
    cid = lax.axis_index("c")
    sid = lax.axis_index("s")
    wid = cid * NS + sid

    def zfill(i, carry):
        z = jnp.zeros((L,), jnp.float32)
        for g in range(D // L):
            vdst[i, pl.ds(g * L, L)] = z
        return carry

    lax.fori_loop(0, CHUNK, zfill, 0)
    rbase = sid * RPW
    for k in range(RPW // CHUNK):
        pltpu.sync_copy(vdst, acc.at[pl.ds(rbase + k * CHUNK, CHUNK)])

    def fill(i, carry):
        z = jnp.zeros((L,), jnp.float32)
        one = z + 1.0
        for g in range(D // L):
            vsrc[i, pl.ds(g * L, L)] = one if g == OUT_LANE // L else z
            if g == IN_LANE // L:
                vdst[i, pl.ds(g * L, L)] = one
        return carry

    lax.fori_loop(0, CHUNK, fill, 0)
    plsc.subcore_barrier()

    ebase = wid * EPW

    def chunk_body(c, carry):
        off = ebase + c * CHUNK
        pltpu.sync_copy(src_hbm.at[pl.ds(off, CHUNK)], src_v)
        pltpu.sync_copy(dst_hbm.at[pl.ds(off, CHUNK)], dst_v)
        pltpu.sync_copy(vsrc, acc.at[src_v], add=True)
        pltpu.sync_copy(vdst, acc.at[dst_v], add=True)
        return carry

    lax.fori_loop(0, CPW, chunk_body, 0)
    plsc.subcore_barrier()

    pltpu.sync_copy(acc.at[pl.ds(rbase, RPW)],
                    deg_hbm.at[cid, pl.ds(rbase, RPW)])


_sc_degrees = pl.kernel(
    _sc_degrees_body,
    out_type=jax.ShapeDtypeStruct((NC, N_PAD, D), jnp.float32),
    mesh=_sc_mesh,
    scratch_types=[
        pltpu.VMEM_SHARED((N_PAD, D), jnp.float32),
        pltpu.VMEM((CHUNK,), jnp.int32),
        pltpu.VMEM((CHUNK,), jnp.int32),
        pltpu.VMEM((CHUNK, D), jnp.float32),
        pltpu.VMEM((CHUNK, D), jnp.float32),
    ],
)


def _tc_pre_body(x_ref, w1_ref, wfc_ref, g_ref, b_ref, dg_ref,
                 h1_ref, f1s_ref, rio_ref, rsin_ref):
    x = x_ref[...]
    deg_o = dg_ref[0, :, OUT_LANE:OUT_LANE + 1] + dg_ref[1, :, OUT_LANE:OUT_LANE + 1]
    deg_i = dg_ref[0, :, IN_LANE:IN_LANE + 1] + dg_ref[1, :, IN_LANE:IN_LANE + 1]
    rs_o = jax.lax.rsqrt(jnp.maximum(deg_o, 1.0))
    rs_i = jax.lax.rsqrt(jnp.maximum(deg_i, 1.0))
    xo = x * rs_o
    h1_ref[...] = jnp.dot(xo, w1_ref[...], preferred_element_type=jnp.float32)
    f = jnp.dot(x, wfc_ref[...], preferred_element_type=jnp.float32)
    mu = jnp.mean(f, axis=-1, keepdims=True)
    var = jnp.mean((f - mu) * (f - mu), axis=-1, keepdims=True)
    f1 = jnp.maximum((f - mu) * jax.lax.rsqrt(var + 1e-5) * g_ref[...] + b_ref[...], 0.0)
    f1s_ref[...] = f1 * rs_o
    rio_ref[...] = rs_i * rs_o
    rsin_ref[...] = rs_i


def _tc_pre(x, W1, Wfc, ln_g, ln_b, degp):
    return pl.pallas_call(
        _tc_pre_body,
        grid=(GRID,),
        in_specs=[
            pl.BlockSpec((BM, D), lambda i: (i, 0)),
            pl.BlockSpec((D, D), lambda i: (0, 0)),
            pl.BlockSpec((D, D), lambda i: (0, 0)),
            pl.BlockSpec((1, D), lambda i: (0, 0)),
            pl.BlockSpec((1, D), lambda i: (0, 0)),
            pl.BlockSpec((NC, BM, D), lambda i: (0, i, 0)),
        ],
        out_specs=[
            pl.BlockSpec((BM, D), lambda i: (i, 0)),
            pl.BlockSpec((BM, D), lambda i: (i, 0)),
            pl.BlockSpec((BM, 1), lambda i: (i, 0)),
            pl.BlockSpec((BM, 1), lambda i: (i, 0)),
        ],
        out_shape=[
            jax.ShapeDtypeStruct((N, D), jnp.float32),
            jax.ShapeDtypeStruct((N, D), jnp.float32),
            jax.ShapeDtypeStruct((N, 1), jnp.float32),
            jax.ShapeDtypeStruct((N, 1), jnp.float32),
        ],
    )(x, W1, Wfc, ln_g.reshape(1, D), ln_b.reshape(1, D), degp)


def _tc_mid2_body(agg_ref, f1s_ref, w2a_ref, w2b_ref, rio_ref, out_ref):
    x1s = jnp.maximum(agg_ref[0] + agg_ref[1], 0.0) * rio_ref[...]
    out_ref[...] = (
        jnp.dot(x1s, w2a_ref[...], preferred_element_type=jnp.float32)
        + jnp.dot(f1s_ref[...], w2b_ref[...], preferred_element_type=jnp.float32))


def _tc_mid2(aggp, f1s, W2a, W2b, rio):
    return pl.pallas_call(
        _tc_mid2_body,
        grid=(GRID,),
        in_specs=[
            pl.BlockSpec((NC, BM, D), lambda i: (0, i, 0)),
            pl.BlockSpec((BM, D), lambda i: (i, 0)),
            pl.BlockSpec((D, D), lambda i: (0, 0)),
            pl.BlockSpec((D, D), lambda i: (0, 0)),
            pl.BlockSpec((BM, 1), lambda i: (i, 0)),
        ],
        out_specs=pl.BlockSpec((BM, D), lambda i: (i, 0)),
        out_shape=jax.ShapeDtypeStruct((N, D), jnp.float32),
    )(aggp, f1s, W2a, W2b, rio)


def _tc_mid3_body(agg_ref, w3_ref, rio_ref, out_ref):
    x2s = jnp.maximum(agg_ref[0] + agg_ref[1], 0.0) * rio_ref[...]
    out_ref[...] = jnp.dot(x2s, w3_ref[...], preferred_element_type=jnp.float32)


def _tc_mid3(aggp, W3, rio):
    return pl.pallas_call(
        _tc_mid3_body,
        grid=(GRID,),
        in_specs=[
            pl.BlockSpec((NC, BM, D), lambda i: (0, i, 0)),
            pl.BlockSpec((D, D), lambda i: (0, 0)),
            pl.BlockSpec((BM, 1), lambda i: (i, 0)),
        ],
        out_specs=pl.BlockSpec((BM, D), lambda i: (i, 0)),
        out_shape=jax.ShapeDtypeStruct((N, D), jnp.float32),
    )(aggp, W3, rio)


def _tc_final_body(agg_ref, rsin_ref, out_ref):
    i = pl.program_id(0)
    x3 = jnp.maximum(agg_ref[0] + agg_ref[1], 0.0) * rsin_ref[...]
    part = jnp.sum(x3, axis=0, keepdims=True)

    @pl.when(i == 0)
    def _():
        out_ref[...] = part

    @pl.when(i != 0)
    def _():
        out_ref[...] += part


def _tc_final(aggp, rs_in):
    return pl.pallas_call(
        _tc_final_body,
        grid=(GRID,),
        in_specs=[
            pl.BlockSpec((NC, BM, D), lambda i: (0, i, 0)),
            pl.BlockSpec((BM, 1), lambda i: (i, 0)),
        ],
        out_specs=pl.BlockSpec((1, D), lambda i: (0, 0)),
        out_shape=jax.ShapeDtypeStruct((1, D), jnp.float32),
    )(aggp, rs_in)


def kernel(x, edge_index, w, W1, Wfc, ln_g, ln_b, W2, W3):
    src = edge_index[0].astype(jnp.int32)
    dst = edge_index[1].astype(jnp.int32)
    pad = E_PAD - E
    zpad_i = jnp.zeros((pad,), jnp.int32)
    src_g = jnp.concatenate([src, zpad_i])
    dst_g = jnp.concatenate([dst, zpad_i])
    w_g = jnp.concatenate([w, jnp.zeros((pad,), jnp.float32)])
    sinkpad = jnp.full((pad,), SINK, jnp.int32)
    src_d = jnp.concatenate([src, sinkpad])
    dst_d = jnp.concatenate([dst, sinkpad])

    degp = _sc_degrees(src_d, dst_d)
    h1, f1s, rio, rs_in = _tc_pre(x, W1, Wfc, ln_g, ln_b, degp)
    agg1 = _sc_aggregate(h1, src_g, dst_g, w_g)
    h2 = _tc_mid2(agg1, f1s, W2[:D], W2[D:], rio)
    agg2 = _sc_aggregate(h2, src_g, dst_g, w_g)
    h3 = _tc_mid3(agg2, W3, rio)
    agg3 = _sc_aggregate(h3, src_g, dst_g, w_g)
    return _tc_final(agg3, rs_in)


# speedup vs baseline: 3.2022x; 1.0790x over previous
import functools

import jax
import jax.numpy as jnp
from jax import lax
from jax.experimental import pallas as pl
from jax.experimental.pallas import tpu as pltpu
from jax.experimental.pallas import tpu_sc as plsc

N = 10000
D = 128
E = 320000
BM = 1000
GRID = N // BM

NC, NS, L = 2, 16, 16
NW = NC * NS
CHUNK = 128
CPW = 80
EPW = CPW * CHUNK
E_PAD = NW * EPW
NB = 2
N_PAD = 10240
RPW = N_PAD // NS
SINK = N

_sc_mesh = plsc.VectorSubcoreMesh(core_axis_name="c", subcore_axis_name="s")


def _sc_aggregate_body(h_hbm, src_hbm, dst_hbm, w_hbm, out_hbm,
                       acc, src0, src1, dst0, dst1, w0, w1,
                       rows0, rows1,
                       sg0, sg1, ss0, ss1, si0, si1):
    srcs, dsts, ws = [src0, src1], [dst0, dst1], [w0, w1]
    rowss = [rows0, rows1]
    sgs, sss, sis = [sg0, sg1], [ss0, ss1], [si0, si1]
    cid = lax.axis_index("c")
    sid = lax.axis_index("s")
    wid = cid * NS + sid

    def zrow(i, carry):
        z = jnp.zeros((L,), jnp.float32)
        for g in range(D // L):
            rows0[i, pl.ds(g * L, L)] = z
        return carry

    lax.fori_loop(0, CHUNK, zrow, 0)
    rbase = sid * RPW
    for k in range(RPW // CHUNK):
        pltpu.sync_copy(rows0, acc.at[pl.ds(rbase + k * CHUNK, CHUNK)])
    plsc.subcore_barrier()

    ebase = wid * EPW
    pltpu.sync_copy(src_hbm.at[pl.ds(ebase, CHUNK)], src0)
    pltpu.sync_copy(dst_hbm.at[pl.ds(ebase, CHUNK)], dst0)
    pltpu.sync_copy(w_hbm.at[pl.ds(ebase, CHUNK)], w0.at[pl.ds(0, CHUNK)])
    pltpu.async_copy(h_hbm.at[src0], rows0, sg0)

    def outer(k0, carry):
        for j in range(NB):
            k = k0 * NB + j
            b, p, b1, p1 = j, j % 2, (j + 1) % NB, (j + 1) % 2
            off1 = ebase + (k + 1) * CHUNK
            more = k + 1 < CPW

            @pl.when(more)
            def _():
                pltpu.async_copy(src_hbm.at[pl.ds(off1, CHUNK)], srcs[p1], sis[p1])
                pltpu.async_copy(dst_hbm.at[pl.ds(off1, CHUNK)], dsts[p1], sis[p1])
                pltpu.async_copy(w_hbm.at[pl.ds(off1, CHUNK)],
                                 ws[p1].at[pl.ds(0, CHUNK)], sis[p1])

            pltpu.make_async_copy(h_hbm.at[srcs[p]], rowss[b], sgs[b]).wait()
            rb, wb = rowss[b], ws[p]

            def mul_body(e, c2):
                wsc = wb[pl.ds(e, L)][0]
                for g in range(D // L):
                    sl = pl.ds(g * L, L)
                    rb[e, sl] = rb[e, sl] * wsc
                return c2

            lax.fori_loop(0, CHUNK, mul_body, 0, unroll=2)

            pltpu.async_copy(rowss[b], acc.at[dsts[p]], sss[b], add=True)

            @pl.when(more)
            def _():
                @pl.when(k >= NB - 1)
                def _():
                    pltpu.make_async_copy(rowss[b1], acc.at[dsts[p1]], sss[b1]).wait()

                pltpu.make_async_copy(src_hbm.at[pl.ds(off1, CHUNK)], srcs[p1], sis[p1]).wait()
                pltpu.make_async_copy(dst_hbm.at[pl.ds(off1, CHUNK)], dsts[p1], sis[p1]).wait()
                pltpu.make_async_copy(w_hbm.at[pl.ds(off1, CHUNK)],
                                      ws[p1].at[pl.ds(0, CHUNK)], sis[p1]).wait()
                pltpu.async_copy(h_hbm.at[srcs[p1]], rowss[b1], sgs[b1])
        return carry

    lax.fori_loop(0, CPW // NB, outer, 0)
    for b in range(NB):
        pltpu.make_async_copy(rowss[b], acc.at[dsts[b % 2]], sss[b]).wait()
    plsc.subcore_barrier()

    pltpu.sync_copy(acc.at[pl.ds(rbase, RPW)],
                    out_hbm.at[cid, pl.ds(rbase, RPW)])


_sc_aggregate = pl.kernel(
    _sc_aggregate_body,
    out_type=jax.ShapeDtypeStruct((NC, N_PAD, D), jnp.float32),
    mesh=_sc_mesh,
    scratch_types=[
        pltpu.VMEM_SHARED((N_PAD, D), jnp.float32),
        pltpu.VMEM((CHUNK,), jnp.int32),
        pltpu.VMEM((CHUNK,), jnp.int32),
        pltpu.VMEM((CHUNK,), jnp.int32),
        pltpu.VMEM((CHUNK,), jnp.int32),
        pltpu.VMEM((CHUNK + L,), jnp.float32),
        pltpu.VMEM((CHUNK + L,), jnp.float32),
        pltpu.VMEM((CHUNK, D), jnp.float32),
        pltpu.VMEM((CHUNK, D), jnp.float32),
        pltpu.SemaphoreType.DMA,
        pltpu.SemaphoreType.DMA,
        pltpu.SemaphoreType.DMA,
        pltpu.SemaphoreType.DMA,
        pltpu.SemaphoreType.DMA,
        pltpu.SemaphoreType.DMA,
    ],
)


OUT_LANE = 0
IN_LANE = 64


def _sc_degrees_body(src_hbm, dst_hbm, deg_hbm,
                     acc, src0, src1, dst0, dst1, vsrc, vdst,
                     ss0, ss1, si0, si1):
    srcs, dsts = [src0, src1], [dst0, dst1]
    sss, sis = [ss0, ss1], [si0, si1]
    cid = lax.axis_index("c")
    sid = lax.axis_index("s")
    wid = cid * NS + sid

    def zfill(i, carry):
        z = jnp.zeros((L,), jnp.float32)
        for g in range(D // L):
            vdst[i, pl.ds(g * L, L)] = z
        return carry

    lax.fori_loop(0, CHUNK, zfill, 0)
    rbase = sid * RPW
    for k in range(RPW // CHUNK):
        pltpu.sync_copy(vdst, acc.at[pl.ds(rbase + k * CHUNK, CHUNK)])

    def fill(i, carry):
        z = jnp.zeros((L,), jnp.float32)
        one = z + 1.0
        for g in range(D // L):
            vsrc[i, pl.ds(g * L, L)] = one if g == OUT_LANE // L else z
            if g == IN_LANE // L:
                vdst[i, pl.ds(g * L, L)] = one
        return carry

    lax.fori_loop(0, CHUNK, fill, 0)
    plsc.subcore_barrier()

    ebase = wid * EPW
    pltpu.sync_copy(src_hbm.at[pl.ds(ebase, CHUNK)], src0)
    pltpu.sync_copy(dst_hbm.at[pl.ds(ebase, CHUNK)], dst0)

    def outer(k0, carry):
        for j in range(2):
            k = k0 * 2 + j
            p, p1 = j, (j + 1) % 2
            off1 = ebase + (k + 1) * CHUNK
            more = k + 1 < CPW

            @pl.when(k >= 1)
            def _():
                pltpu.make_async_copy(vsrc, acc.at[srcs[p1]], sss[p1]).wait()
                pltpu.make_async_copy(vdst, acc.at[dsts[p1]], sss[p1]).wait()

            @pl.when(more)
            def _():
                pltpu.async_copy(src_hbm.at[pl.ds(off1, CHUNK)], srcs[p1], sis[p1])
                pltpu.async_copy(dst_hbm.at[pl.ds(off1, CHUNK)], dsts[p1], sis[p1])

            pltpu.async_copy(vsrc, acc.at[srcs[p]], sss[p], add=True)
            pltpu.async_copy(vdst, acc.at[dsts[p]], sss[p], add=True)

            @pl.when(more)
            def _():
                pltpu.make_async_copy(src_hbm.at[pl.ds(off1, CHUNK)], srcs[p1], sis[p1]).wait()
                pltpu.make_async_copy(dst_hbm.at[pl.ds(off1, CHUNK)], dsts[p1], sis[p1]).wait()
        return carry

    lax.fori_loop(0, CPW // 2, outer, 0)
    pltpu.make_async_copy(vsrc, acc.at[srcs[1]], sss[1]).wait()
    pltpu.make_async_copy(vdst, acc.at[dsts[1]], sss[1]).wait()
    plsc.subcore_barrier()

    pltpu.sync_copy(acc.at[pl.ds(rbase, RPW)],
                    deg_hbm.at[cid, pl.ds(rbase, RPW)])


_sc_degrees = pl.kernel(
    _sc_degrees_body,
    out_type=jax.ShapeDtypeStruct((NC, N_PAD, D), jnp.float32),
    mesh=_sc_mesh,
    scratch_types=[
        pltpu.VMEM_SHARED((N_PAD, D), jnp.float32),
        pltpu.VMEM((CHUNK,), jnp.int32),
        pltpu.VMEM((CHUNK,), jnp.int32),
        pltpu.VMEM((CHUNK,), jnp.int32),
        pltpu.VMEM((CHUNK,), jnp.int32),
        pltpu.VMEM((CHUNK, D), jnp.float32),
        pltpu.VMEM((CHUNK, D), jnp.float32),
        pltpu.SemaphoreType.DMA,
        pltpu.SemaphoreType.DMA,
        pltpu.SemaphoreType.DMA,
        pltpu.SemaphoreType.DMA,
    ],
)


def _tc_matmuls_body(x_ref, w1_ref, wfc_ref, g_ref, b_ref, xw1_ref, f1_ref):
    x = x_ref[...]
    xw1_ref[...] = jnp.dot(x, w1_ref[...], preferred_element_type=jnp.float32)
    f = jnp.dot(x, wfc_ref[...], preferred_element_type=jnp.float32)
    mu = jnp.mean(f, axis=-1, keepdims=True)
    var = jnp.mean((f - mu) * (f - mu), axis=-1, keepdims=True)
    f1_ref[...] = jnp.maximum(
        (f - mu) * jax.lax.rsqrt(var + 1e-5) * g_ref[...] + b_ref[...], 0.0)


def _tc_matmuls(x, W1, Wfc, ln_g, ln_b):
    return pl.pallas_call(
        _tc_matmuls_body,
        grid=(GRID,),
        in_specs=[
            pl.BlockSpec((BM, D), lambda i: (i, 0)),
            pl.BlockSpec((D, D), lambda i: (0, 0)),
            pl.BlockSpec((D, D), lambda i: (0, 0)),
            pl.BlockSpec((1, D), lambda i: (0, 0)),
            pl.BlockSpec((1, D), lambda i: (0, 0)),
        ],
        out_specs=[
            pl.BlockSpec((BM, D), lambda i: (i, 0)),
            pl.BlockSpec((BM, D), lambda i: (i, 0)),
        ],
        out_shape=[
            jax.ShapeDtypeStruct((N, D), jnp.float32),
            jax.ShapeDtypeStruct((N, D), jnp.float32),
        ],
    )(x, W1, Wfc, ln_g.reshape(1, D), ln_b.reshape(1, D))


def _tc_scale_body(xw1_ref, f1_ref, dg_ref, h1_ref, f1s_ref, rio_ref, rsin_ref):
    deg_o = dg_ref[0, :, OUT_LANE:OUT_LANE + 1] + dg_ref[1, :, OUT_LANE:OUT_LANE + 1]
    deg_i = dg_ref[0, :, IN_LANE:IN_LANE + 1] + dg_ref[1, :, IN_LANE:IN_LANE + 1]
    rs_o = jax.lax.rsqrt(jnp.maximum(deg_o, 1.0))
    rs_i = jax.lax.rsqrt(jnp.maximum(deg_i, 1.0))
    h1_ref[...] = xw1_ref[...] * rs_o
    f1s_ref[...] = f1_ref[...] * rs_o
    rio_ref[...] = rs_i * rs_o
    rsin_ref[...] = rs_i


def _tc_scale(xw1, f1, degp):
    return pl.pallas_call(
        _tc_scale_body,
        grid=(GRID,),
        in_specs=[
            pl.BlockSpec((BM, D), lambda i: (i, 0)),
            pl.BlockSpec((BM, D), lambda i: (i, 0)),
            pl.BlockSpec((NC, BM, D), lambda i: (0, i, 0)),
        ],
        out_specs=[
            pl.BlockSpec((BM, D), lambda i: (i, 0)),
            pl.BlockSpec((BM, D), lambda i: (i, 0)),
            pl.BlockSpec((BM, 1), lambda i: (i, 0)),
            pl.BlockSpec((BM, 1), lambda i: (i, 0)),
        ],
        out_shape=[
            jax.ShapeDtypeStruct((N, D), jnp.float32),
            jax.ShapeDtypeStruct((N, D), jnp.float32),
            jax.ShapeDtypeStruct((N, 1), jnp.float32),
            jax.ShapeDtypeStruct((N, 1), jnp.float32),
        ],
    )(xw1, f1, degp)


def _tc_mid2_body(agg_ref, f1s_ref, w2a_ref, w2b_ref, rio_ref, out_ref):
    x1s = jnp.maximum(agg_ref[0] + agg_ref[1], 0.0) * rio_ref[...]
    out_ref[...] = (
        jnp.dot(x1s, w2a_ref[...], preferred_element_type=jnp.float32)
        + jnp.dot(f1s_ref[...], w2b_ref[...], preferred_element_type=jnp.float32))


def _tc_mid2(aggp, f1s, W2a, W2b, rio):
    return pl.pallas_call(
        _tc_mid2_body,
        grid=(GRID,),
        in_specs=[
            pl.BlockSpec((NC, BM, D), lambda i: (0, i, 0)),
            pl.BlockSpec((BM, D), lambda i: (i, 0)),
            pl.BlockSpec((D, D), lambda i: (0, 0)),
            pl.BlockSpec((D, D), lambda i: (0, 0)),
            pl.BlockSpec((BM, 1), lambda i: (i, 0)),
        ],
        out_specs=pl.BlockSpec((BM, D), lambda i: (i, 0)),
        out_shape=jax.ShapeDtypeStruct((N, D), jnp.float32),
    )(aggp, f1s, W2a, W2b, rio)


def _tc_mid3_body(agg_ref, w3_ref, rio_ref, out_ref):
    x2s = jnp.maximum(agg_ref[0] + agg_ref[1], 0.0) * rio_ref[...]
    out_ref[...] = jnp.dot(x2s, w3_ref[...], preferred_element_type=jnp.float32)


def _tc_mid3(aggp, W3, rio):
    return pl.pallas_call(
        _tc_mid3_body,
        grid=(GRID,),
        in_specs=[
            pl.BlockSpec((NC, BM, D), lambda i: (0, i, 0)),
            pl.BlockSpec((D, D), lambda i: (0, 0)),
            pl.BlockSpec((BM, 1), lambda i: (i, 0)),
        ],
        out_specs=pl.BlockSpec((BM, D), lambda i: (i, 0)),
        out_shape=jax.ShapeDtypeStruct((N, D), jnp.float32),
    )(aggp, W3, rio)


def _tc_final_body(agg_ref, rsin_ref, out_ref):
    i = pl.program_id(0)
    x3 = jnp.maximum(agg_ref[0] + agg_ref[1], 0.0) * rsin_ref[...]
    part = jnp.sum(x3, axis=0, keepdims=True)

    @pl.when(i == 0)
    def _():
        out_ref[...] = part

    @pl.when(i != 0)
    def _():
        out_ref[...] += part


def _tc_final(aggp, rs_in):
    return pl.pallas_call(
        _tc_final_body,
        grid=(GRID,),
        in_specs=[
            pl.BlockSpec((NC, BM, D), lambda i: (0, i, 0)),
            pl.BlockSpec((BM, 1), lambda i: (i, 0)),
        ],
        out_specs=pl.BlockSpec((1, D), lambda i: (0, 0)),
        out_shape=jax.ShapeDtypeStruct((1, D), jnp.float32),
    )(aggp, rs_in)


def kernel(x, edge_index, w, W1, Wfc, ln_g, ln_b, W2, W3):
    src = edge_index[0].astype(jnp.int32)
    dst = edge_index[1].astype(jnp.int32)
    pad = E_PAD - E
    zpad_i = jnp.zeros((pad,), jnp.int32)
    src_g = jnp.concatenate([src, zpad_i])
    dst_g = jnp.concatenate([dst, zpad_i])
    w_g = jnp.concatenate([w, jnp.zeros((pad,), jnp.float32)])
    sinkpad = jnp.full((pad,), SINK, jnp.int32)
    src_d = jnp.concatenate([src, sinkpad])
    dst_d = jnp.concatenate([dst, sinkpad])

    degp = _sc_degrees(src_d, dst_d)
    xw1, f1 = _tc_matmuls(x, W1, Wfc, ln_g, ln_b)
    h1, f1s, rio, rs_in = _tc_scale(xw1, f1, degp)
    agg1 = _sc_aggregate(h1, src_g, dst_g, w_g)
    h2 = _tc_mid2(agg1, f1s, W2[:D], W2[D:], rio)
    agg2 = _sc_aggregate(h2, src_g, dst_g, w_g)
    h3 = _tc_mid3(agg2, W3, rio)
    agg3 = _sc_aggregate(h3, src_g, dst_g, w_g)
    return _tc_final(agg3, rs_in)


# speedup vs baseline: 5.8011x; 1.8116x over previous
import functools

import jax
import jax.numpy as jnp
from jax import lax
from jax.experimental import pallas as pl
from jax.experimental.pallas import tpu as pltpu
from jax.experimental.pallas import tpu_sc as plsc

N = 10000
D = 128
E = 320000
BM = 1000
GRID = N // BM

NC, NS, L = 2, 16, 16
NW = NC * NS
CHUNK = 128
CPW = 80
EPW = CPW * CHUNK
E_PAD = NW * EPW
NB = 2
N_PAD = 10240
RPW = N_PAD // NS
SINK = N

_sc_mesh = plsc.VectorSubcoreMesh(core_axis_name="c", subcore_axis_name="s")


DH = D // NC
CPT = E_PAD // (NS * CHUNK)
EPT = CPT * CHUNK


def _sc_aggregate_body(h_hbm, src_hbm, dst_hbm, w_hbm, out_hbm,
                       hstage, acc, src0, src1, dst0, dst1, w0, w1,
                       rows0, rows1,
                       sg0, sg1, ss0, ss1, si0, si1, sh):
    srcs, dsts, ws = [src0, src1], [dst0, dst1], [w0, w1]
    rowss = [rows0, rows1]
    sgs, sss, sis = [sg0, sg1], [ss0, ss1], [si0, si1]
    cid = lax.axis_index("c")
    sid = lax.axis_index("s")

    rbase = sid * RPW
    pltpu.async_copy(h_hbm.at[cid, pl.ds(rbase, RPW)],
                     hstage.at[pl.ds(rbase, RPW)], sh)

    def zrow(i, carry):
        z = jnp.zeros((L,), jnp.float32)
        for g in range(DH // L):
            rows0[i, pl.ds(g * L, L)] = z
        return carry

    lax.fori_loop(0, CHUNK, zrow, 0)
    for k in range(RPW // CHUNK):
        pltpu.sync_copy(rows0, acc.at[pl.ds(rbase + k * CHUNK, CHUNK)])
    pltpu.make_async_copy(h_hbm.at[cid, pl.ds(rbase, RPW)],
                          hstage.at[pl.ds(rbase, RPW)], sh).wait()
    plsc.subcore_barrier()

    ebase = sid * EPT
    pltpu.sync_copy(src_hbm.at[pl.ds(ebase, CHUNK)], src0)
    pltpu.sync_copy(dst_hbm.at[pl.ds(ebase, CHUNK)], dst0)
    pltpu.sync_copy(w_hbm.at[pl.ds(ebase, CHUNK)], w0.at[pl.ds(0, CHUNK)])
    pltpu.async_copy(hstage.at[src0], rows0, sg0)

    def outer(k0, carry):
        for j in range(NB):
            k = k0 * NB + j
            b, p, b1, p1 = j, j % 2, (j + 1) % NB, (j + 1) % 2
            off1 = ebase + (k + 1) * CHUNK
            more = k + 1 < CPT

            @pl.when(more)
            def _():
                pltpu.async_copy(src_hbm.at[pl.ds(off1, CHUNK)], srcs[p1], sis[p1])
                pltpu.async_copy(dst_hbm.at[pl.ds(off1, CHUNK)], dsts[p1], sis[p1])
                pltpu.async_copy(w_hbm.at[pl.ds(off1, CHUNK)],
                                 ws[p1].at[pl.ds(0, CHUNK)], sis[p1])

            pltpu.make_async_copy(hstage.at[srcs[p]], rowss[b], sgs[b]).wait()
            rb, wb = rowss[b], ws[p]

            def mul_body(e, c2):
                wsc = wb[pl.ds(e, L)][0]
                for g in range(DH // L):
                    sl = pl.ds(g * L, L)
                    rb[e, sl] = rb[e, sl] * wsc
                return c2

            lax.fori_loop(0, CHUNK, mul_body, 0, unroll=4)

            pltpu.async_copy(rowss[b], acc.at[dsts[p]], sss[b], add=True)

            @pl.when(more)
            def _():
                @pl.when(k >= NB - 1)
                def _():
                    pltpu.make_async_copy(rowss[b1], acc.at[dsts[p1]], sss[b1]).wait()

                pltpu.make_async_copy(src_hbm.at[pl.ds(off1, CHUNK)], srcs[p1], sis[p1]).wait()
                pltpu.make_async_copy(dst_hbm.at[pl.ds(off1, CHUNK)], dsts[p1], sis[p1]).wait()
                pltpu.make_async_copy(w_hbm.at[pl.ds(off1, CHUNK)],
                                      ws[p1].at[pl.ds(0, CHUNK)], sis[p1]).wait()
                pltpu.async_copy(hstage.at[srcs[p1]], rowss[b1], sgs[b1])
        return carry

    lax.fori_loop(0, CPT // NB, outer, 0)
    for b in range(NB):
        pltpu.make_async_copy(rowss[b], acc.at[dsts[b % 2]], sss[b]).wait()
    plsc.subcore_barrier()

    pltpu.sync_copy(acc.at[pl.ds(rbase, RPW)],
                    out_hbm.at[cid, pl.ds(rbase, RPW)])


_sc_aggregate = pl.kernel(
    _sc_aggregate_body,
    out_type=jax.ShapeDtypeStruct((NC, N_PAD, DH), jnp.float32),
    mesh=_sc_mesh,
    scratch_types=[
        pltpu.VMEM_SHARED((N_PAD, DH), jnp.float32),
        pltpu.VMEM_SHARED((N_PAD, DH), jnp.float32),
        pltpu.VMEM((CHUNK,), jnp.int32),
        pltpu.VMEM((CHUNK,), jnp.int32),
        pltpu.VMEM((CHUNK,), jnp.int32),
        pltpu.VMEM((CHUNK,), jnp.int32),
        pltpu.VMEM((CHUNK + L,), jnp.float32),
        pltpu.VMEM((CHUNK + L,), jnp.float32),
        pltpu.VMEM((CHUNK, DH), jnp.float32),
        pltpu.VMEM((CHUNK, DH), jnp.float32),
        pltpu.SemaphoreType.DMA,
        pltpu.SemaphoreType.DMA,
        pltpu.SemaphoreType.DMA,
        pltpu.SemaphoreType.DMA,
        pltpu.SemaphoreType.DMA,
        pltpu.SemaphoreType.DMA,
        pltpu.SemaphoreType.DMA,
    ],
)


OUT_LANE = 0
IN_LANE = 64


def _sc_degrees_body(src_hbm, dst_hbm, deg_hbm,
                     acc, src0, src1, dst0, dst1, vsrc, vdst,
                     ss0, ss1, si0, si1):
    srcs, dsts = [src0, src1], [dst0, dst1]
    sss, sis = [ss0, ss1], [si0, si1]
    cid = lax.axis_index("c")
    sid = lax.axis_index("s")
    wid = cid * NS + sid

    def zfill(i, carry):
        z = jnp.zeros((L,), jnp.float32)
        for g in range(D // L):
            vdst[i, pl.ds(g * L, L)] = z
        return carry

    lax.fori_loop(0, CHUNK, zfill, 0)
    rbase = sid * RPW
    for k in range(RPW // CHUNK):
        pltpu.sync_copy(vdst, acc.at[pl.ds(rbase + k * CHUNK, CHUNK)])

    def fill(i, carry):
        z = jnp.zeros((L,), jnp.float32)
        one = z + 1.0
        for g in range(D // L):
            vsrc[i, pl.ds(g * L, L)] = one if g == OUT_LANE // L else z
            if g == IN_LANE // L:
                vdst[i, pl.ds(g * L, L)] = one
        return carry

    lax.fori_loop(0, CHUNK, fill, 0)
    plsc.subcore_barrier()

    ebase = wid * EPW
    pltpu.sync_copy(src_hbm.at[pl.ds(ebase, CHUNK)], src0)
    pltpu.sync_copy(dst_hbm.at[pl.ds(ebase, CHUNK)], dst0)

    def outer(k0, carry):
        for j in range(2):
            k = k0 * 2 + j
            p, p1 = j, (j + 1) % 2
            off1 = ebase + (k + 1) * CHUNK
            more = k + 1 < CPW

            @pl.when(k >= 1)
            def _():
                pltpu.make_async_copy(vsrc, acc.at[srcs[p1]], sss[p1]).wait()
                pltpu.make_async_copy(vdst, acc.at[dsts[p1]], sss[p1]).wait()

            @pl.when(more)
            def _():
                pltpu.async_copy(src_hbm.at[pl.ds(off1, CHUNK)], srcs[p1], sis[p1])
                pltpu.async_copy(dst_hbm.at[pl.ds(off1, CHUNK)], dsts[p1], sis[p1])

            pltpu.async_copy(vsrc, acc.at[srcs[p]], sss[p], add=True)
            pltpu.async_copy(vdst, acc.at[dsts[p]], sss[p], add=True)

            @pl.when(more)
            def _():
                pltpu.make_async_copy(src_hbm.at[pl.ds(off1, CHUNK)], srcs[p1], sis[p1]).wait()
                pltpu.make_async_copy(dst_hbm.at[pl.ds(off1, CHUNK)], dsts[p1], sis[p1]).wait()
        return carry

    lax.fori_loop(0, CPW // 2, outer, 0)
    pltpu.make_async_copy(vsrc, acc.at[srcs[1]], sss[1]).wait()
    pltpu.make_async_copy(vdst, acc.at[dsts[1]], sss[1]).wait()
    plsc.subcore_barrier()

    pltpu.sync_copy(acc.at[pl.ds(rbase, RPW)],
                    deg_hbm.at[cid, pl.ds(rbase, RPW)])


_sc_degrees = pl.kernel(
    _sc_degrees_body,
    out_type=jax.ShapeDtypeStruct((NC, N_PAD, D), jnp.float32),
    mesh=_sc_mesh,
    scratch_types=[
        pltpu.VMEM_SHARED((N_PAD, D), jnp.float32),
        pltpu.VMEM((CHUNK,), jnp.int32),
        pltpu.VMEM((CHUNK,), jnp.int32),
        pltpu.VMEM((CHUNK,), jnp.int32),
        pltpu.VMEM((CHUNK,), jnp.int32),
        pltpu.VMEM((CHUNK, D), jnp.float32),
        pltpu.VMEM((CHUNK, D), jnp.float32),
        pltpu.SemaphoreType.DMA,
        pltpu.SemaphoreType.DMA,
        pltpu.SemaphoreType.DMA,
        pltpu.SemaphoreType.DMA,
    ],
)


def _tc_matmuls_body(x_ref, w1_ref, wfc_ref, g_ref, b_ref, xw1_ref, f1_ref):
    x = x_ref[...]
    xw1_ref[...] = jnp.dot(x, w1_ref[...], preferred_element_type=jnp.float32)
    f = jnp.dot(x, wfc_ref[...], preferred_element_type=jnp.float32)
    mu = jnp.mean(f, axis=-1, keepdims=True)
    var = jnp.mean((f - mu) * (f - mu), axis=-1, keepdims=True)
    f1_ref[...] = jnp.maximum(
        (f - mu) * jax.lax.rsqrt(var + 1e-5) * g_ref[...] + b_ref[...], 0.0)


def _tc_matmuls(x, W1, Wfc, ln_g, ln_b):
    return pl.pallas_call(
        _tc_matmuls_body,
        grid=(GRID,),
        in_specs=[
            pl.BlockSpec((BM, D), lambda i: (i, 0)),
            pl.BlockSpec((D, D), lambda i: (0, 0)),
            pl.BlockSpec((D, D), lambda i: (0, 0)),
            pl.BlockSpec((1, D), lambda i: (0, 0)),
            pl.BlockSpec((1, D), lambda i: (0, 0)),
        ],
        out_specs=[
            pl.BlockSpec((BM, D), lambda i: (i, 0)),
            pl.BlockSpec((BM, D), lambda i: (i, 0)),
        ],
        out_shape=[
            jax.ShapeDtypeStruct((N, D), jnp.float32),
            jax.ShapeDtypeStruct((N, D), jnp.float32),
        ],
    )(x, W1, Wfc, ln_g.reshape(1, D), ln_b.reshape(1, D))


def _split(h):
    return h[:, :DH], h[:, DH:]


def _tc_scale_body(xw1_ref, f1_ref, dg_ref, h1_ref, f1s_ref, rio_ref, rsin_ref):
    deg_o = dg_ref[0, :, OUT_LANE:OUT_LANE + 1] + dg_ref[1, :, OUT_LANE:OUT_LANE + 1]
    deg_i = dg_ref[0, :, IN_LANE:IN_LANE + 1] + dg_ref[1, :, IN_LANE:IN_LANE + 1]
    rs_o = jax.lax.rsqrt(jnp.maximum(deg_o, 1.0))
    rs_i = jax.lax.rsqrt(jnp.maximum(deg_i, 1.0))
    h1 = xw1_ref[...] * rs_o
    h1_ref[0], h1_ref[1] = _split(h1)
    f1s_ref[...] = f1_ref[...] * rs_o
    rio_ref[...] = rs_i * rs_o
    rsin_ref[...] = rs_i


def _tc_scale(xw1, f1, degp):
    return pl.pallas_call(
        _tc_scale_body,
        grid=(GRID,),
        in_specs=[
            pl.BlockSpec((BM, D), lambda i: (i, 0)),
            pl.BlockSpec((BM, D), lambda i: (i, 0)),
            pl.BlockSpec((NC, BM, D), lambda i: (0, i, 0)),
        ],
        out_specs=[
            pl.BlockSpec((NC, BM, DH), lambda i: (0, i, 0)),
            pl.BlockSpec((BM, D), lambda i: (i, 0)),
            pl.BlockSpec((BM, 1), lambda i: (i, 0)),
            pl.BlockSpec((BM, 1), lambda i: (i, 0)),
        ],
        out_shape=[
            jax.ShapeDtypeStruct((NC, N_PAD, DH), jnp.float32),
            jax.ShapeDtypeStruct((N, D), jnp.float32),
            jax.ShapeDtypeStruct((N, 1), jnp.float32),
            jax.ShapeDtypeStruct((N, 1), jnp.float32),
        ],
    )(xw1, f1, degp)


def _tc_mid2_body(agg_ref, f1s_ref, w2a_ref, w2b_ref, rio_ref, out_ref):
    rio = rio_ref[...]
    x1s = jnp.concatenate(
        [jnp.maximum(agg_ref[0], 0.0) * rio, jnp.maximum(agg_ref[1], 0.0) * rio],
        axis=1)
    h2 = (jnp.dot(x1s, w2a_ref[...], preferred_element_type=jnp.float32)
          + jnp.dot(f1s_ref[...], w2b_ref[...], preferred_element_type=jnp.float32))
    out_ref[0], out_ref[1] = _split(h2)


def _tc_mid2(aggs, f1s, W2a, W2b, rio):
    return pl.pallas_call(
        _tc_mid2_body,
        grid=(GRID,),
        in_specs=[
            pl.BlockSpec((NC, BM, DH), lambda i: (0, i, 0)),
            pl.BlockSpec((BM, D), lambda i: (i, 0)),
            pl.BlockSpec((D, D), lambda i: (0, 0)),
            pl.BlockSpec((D, D), lambda i: (0, 0)),
            pl.BlockSpec((BM, 1), lambda i: (i, 0)),
        ],
        out_specs=pl.BlockSpec((NC, BM, DH), lambda i: (0, i, 0)),
        out_shape=jax.ShapeDtypeStruct((NC, N_PAD, DH), jnp.float32),
    )(aggs, f1s, W2a, W2b, rio)


def _tc_mid3_body(agg_ref, w3_ref, rio_ref, out_ref):
    rio = rio_ref[...]
    x2s = jnp.concatenate(
        [jnp.maximum(agg_ref[0], 0.0) * rio, jnp.maximum(agg_ref[1], 0.0) * rio],
        axis=1)
    h3 = jnp.dot(x2s, w3_ref[...], preferred_element_type=jnp.float32)
    out_ref[0], out_ref[1] = _split(h3)


def _tc_mid3(aggs, W3, rio):
    return pl.pallas_call(
        _tc_mid3_body,
        grid=(GRID,),
        in_specs=[
            pl.BlockSpec((NC, BM, DH), lambda i: (0, i, 0)),
            pl.BlockSpec((D, D), lambda i: (0, 0)),
            pl.BlockSpec((BM, 1), lambda i: (i, 0)),
        ],
        out_specs=pl.BlockSpec((NC, BM, DH), lambda i: (0, i, 0)),
        out_shape=jax.ShapeDtypeStruct((NC, N_PAD, DH), jnp.float32),
    )(aggs, W3, rio)


def _tc_final_body(agg_ref, rsin_ref, out_ref):
    i = pl.program_id(0)
    rs = rsin_ref[...]
    pa = jnp.sum(jnp.maximum(agg_ref[0], 0.0) * rs, axis=0, keepdims=True)
    pb = jnp.sum(jnp.maximum(agg_ref[1], 0.0) * rs, axis=0, keepdims=True)
    part = jnp.concatenate([pa, pb], axis=1)

    @pl.when(i == 0)
    def _():
        out_ref[...] = part

    @pl.when(i != 0)
    def _():
        out_ref[...] += part


def _tc_final(aggs, rs_in):
    return pl.pallas_call(
        _tc_final_body,
        grid=(GRID,),
        in_specs=[
            pl.BlockSpec((NC, BM, DH), lambda i: (0, i, 0)),
            pl.BlockSpec((BM, 1), lambda i: (i, 0)),
        ],
        out_specs=pl.BlockSpec((1, D), lambda i: (0, 0)),
        out_shape=jax.ShapeDtypeStruct((1, D), jnp.float32),
    )(aggs, rs_in)


def kernel(x, edge_index, w, W1, Wfc, ln_g, ln_b, W2, W3):
    src = edge_index[0].astype(jnp.int32)
    dst = edge_index[1].astype(jnp.int32)
    pad = E_PAD - E
    zpad_i = jnp.zeros((pad,), jnp.int32)
    src_g = jnp.concatenate([src, zpad_i])
    dst_g = jnp.concatenate([dst, zpad_i])
    w_g = jnp.concatenate([w, jnp.zeros((pad,), jnp.float32)])
    sinkpad = jnp.full((pad,), SINK, jnp.int32)
    src_d = jnp.concatenate([src, sinkpad])
    dst_d = jnp.concatenate([dst, sinkpad])

    degp = _sc_degrees(src_d, dst_d)
    xw1, f1 = _tc_matmuls(x, W1, Wfc, ln_g, ln_b)
    h1s, f1s, rio, rs_in = _tc_scale(xw1, f1, degp)
    agg1 = _sc_aggregate(h1s, src_g, dst_g, w_g)
    h2s = _tc_mid2(agg1, f1s, W2[:D], W2[D:], rio)
    agg2 = _sc_aggregate(h2s, src_g, dst_g, w_g)
    h3s = _tc_mid3(agg2, W3, rio)
    agg3 = _sc_aggregate(h3s, src_g, dst_g, w_g)
    return _tc_final(agg3, rs_in)
